# baseline (speedup 1.0000x reference)
import functools

import jax
import jax.numpy as jnp
from jax import lax
from jax.experimental import pallas as pl
from jax.experimental.pallas import tpu as pltpu
from jax.experimental.pallas import tpu_sc as plsc

_B, _L, _D = 16384, 200, 64
_BW = 512
_BT = 128
_NBT = _BW // _BT
_NOB = 3


@functools.lru_cache(maxsize=1)
def _build():
    info = plsc.get_sparse_core_info()
    nw = info.num_cores * info.num_subcores
    assert _B // nw == _BW

    mesh = plsc.VectorSubcoreMesh(core_axis_name="c", subcore_axis_name="s")

    @functools.partial(
        pl.kernel,
        out_type=jax.ShapeDtypeStruct((_D, _L, _B), jnp.float32),
        mesh=mesh,
        scratch_types=[
            pltpu.VMEM((_D * 16,), jnp.float32),
            pltpu.VMEM((_D * 16,), jnp.float32),
            pltpu.VMEM((_L, _BT), jnp.int32),
            pltpu.VMEM((_L, _BT), jnp.float32),
            pltpu.VMEM((_NOB, _L, _BT), jnp.float32),
            [pltpu.SemaphoreType.DMA] * _NOB,
        ],
    )
    def k(xt_hbm, r0s_hbm, drs_hbm, out_hbm,
          r0s_v, drs_v, xb_v, bit_v, obuf_v, osems):
        c = lax.axis_index("c")
        s = lax.axis_index("s")
        wid = s * info.num_cores + c
        bbase = pl.multiple_of(wid * _BW, _BW)

        pltpu.sync_copy(r0s_hbm, r0s_v)
        pltpu.sync_copy(drs_hbm, drs_v)

        dummy_dst = out_hbm.at[0, pl.ds(0, _L), pl.ds(0, _BT)]

        def blk_wait(q):
            pltpu.make_async_copy(obuf_v.at[q], dummy_dst, osems[q]).wait()

        def blk_emit(d_idx, bcol, q, skip_wait):
            if not skip_wait:
                blk_wait(q)
            r0 = r0s_v[pl.ds(d_idx * 16, 16)]
            dr = drs_v[pl.ds(d_idx * 16, 16)]

            def lp_body(lp, cc):
                for j in range(_BT // 16):
                    o = j * 16
                    obuf_v[q, lp, pl.ds(o, 16)] = (
                        bit_v[lp, pl.ds(o, 16)] * dr + r0)
                return cc

            lax.fori_loop(0, _L, lp_body, 0, unroll=False)
            pltpu.async_copy(
                obuf_v.at[q],
                out_hbm.at[d_idx, pl.ds(0, _L), pl.ds(bcol, _BT)],
                osems[q])

        for bt in range(_NBT):
            bcol = pl.multiple_of(bbase + bt * _BT, _BT)
            pltpu.sync_copy(xt_hbm.at[pl.ds(0, _L), pl.ds(bcol, _BT)], xb_v)

            def bin_body(lp, cc):
                for j in range(_BT // 16):
                    o = j * 16
                    bit_v[lp, pl.ds(o, 16)] = (
                        jnp.maximum(
                            1 - jnp.abs(xb_v[lp, pl.ds(o, 16)] - 1), 0)
                        .astype(jnp.float32))
                return cc

            lax.fori_loop(0, _L, bin_body, 0, unroll=False)

            def dd_body(dd, carry):
                d2 = dd * _NOB
                for q in range(_NOB):
                    blk_emit(d2 + q, bcol, q, skip_wait=False)
                return carry

            if bt == 0:
                for q in range(_NOB):
                    blk_emit(q, bcol, q, skip_wait=True)
                lax.fori_loop(1, _D // _NOB, dd_body, 0, unroll=False)
            else:
                lax.fori_loop(0, _D // _NOB, dd_body, 0, unroll=False)
            blk_emit(_D - 1, bcol, (_D - 1) % _NOB, skip_wait=False)

        for q in range(_NOB):
            blk_wait(q)

    return k


def kernel(x, table):
    t0 = table[0]
    r0s = jnp.repeat(t0, 16)
    drs = jnp.repeat(table[1] - t0, 16)
    out3 = _build()(x.T, r0s, drs)
    return jnp.transpose(out3, (2, 0, 1))


# speedup vs baseline: 1.0069x; 1.0069x over previous
import functools

import jax
import jax.numpy as jnp
from jax import lax
from jax.experimental import pallas as pl
from jax.experimental.pallas import tpu as pltpu
from jax.experimental.pallas import tpu_sc as plsc

_B, _L, _D = 16384, 200, 64
_BW = 512
_BT = 128
_NBT = _BW // _BT
_NOB = 2


@functools.lru_cache(maxsize=1)
def _build():
    info = plsc.get_sparse_core_info()
    nw = info.num_cores * info.num_subcores
    assert _B // nw == _BW

    mesh = plsc.VectorSubcoreMesh(core_axis_name="c", subcore_axis_name="s")

    @functools.partial(
        pl.kernel,
        out_type=jax.ShapeDtypeStruct((_D, _L, _B), jnp.float32),
        mesh=mesh,
        scratch_types=[
            pltpu.VMEM((_D * 16,), jnp.float32),
            pltpu.VMEM((_D * 16,), jnp.float32),
            pltpu.VMEM((_L, _BT), jnp.int32),
            pltpu.VMEM((_L, _BT), jnp.float32),
            pltpu.VMEM((_NOB, _L, _BT), jnp.float32),
            [pltpu.SemaphoreType.DMA] * _NOB,
        ],
    )
    def k(xt_hbm, r0s_hbm, drs_hbm, out_hbm,
          r0s_v, drs_v, xb_v, bit_v, obuf_v, osems):
        c = lax.axis_index("c")
        s = lax.axis_index("s")
        wid = s * info.num_cores + c
        bbase = pl.multiple_of(wid * _BW, _BW)

        pltpu.sync_copy(r0s_hbm, r0s_v)
        pltpu.sync_copy(drs_hbm, drs_v)

        dummy_dst = out_hbm.at[0, pl.ds(0, _L), pl.ds(0, _BT)]

        def blk_wait(q):
            pltpu.make_async_copy(obuf_v.at[q], dummy_dst, osems[q]).wait()

        def blk_emit(d_idx, bcol, q, skip_wait):
            if not skip_wait:
                blk_wait(q)
            r0 = r0s_v[pl.ds(d_idx * 16, 16)]
            dr = drs_v[pl.ds(d_idx * 16, 16)]

            def lp_body(lp, cc):
                for j in range(_BT // 16):
                    o = j * 16
                    obuf_v[q, lp, pl.ds(o, 16)] = (
                        bit_v[lp, pl.ds(o, 16)] * dr + r0)
                return cc

            lax.fori_loop(0, _L, lp_body, 0, unroll=False)
            pltpu.async_copy(
                obuf_v.at[q],
                out_hbm.at[d_idx, pl.ds(0, _L), pl.ds(bcol, _BT)],
                osems[q])

        for bt in range(_NBT):
            bcol = pl.multiple_of(bbase + bt * _BT, _BT)
            pltpu.sync_copy(xt_hbm.at[pl.ds(0, _L), pl.ds(bcol, _BT)], xb_v)

            def bin_body(lp, cc):
                for j in range(_BT // 16):
                    o = j * 16
                    bit_v[lp, pl.ds(o, 16)] = (
                        jnp.maximum(
                            1 - jnp.abs(xb_v[lp, pl.ds(o, 16)] - 1), 0)
                        .astype(jnp.float32))
                return cc

            lax.fori_loop(0, _L, bin_body, 0, unroll=False)

            def dd_body(dd, carry):
                d2 = dd * _NOB
                for q in range(_NOB):
                    blk_emit(d2 + q, bcol, q, skip_wait=False)
                return carry

            if bt == 0:
                for q in range(_NOB):
                    blk_emit(q, bcol, q, skip_wait=True)
                lax.fori_loop(1, _D // _NOB, dd_body, 0, unroll=False)
            else:
                lax.fori_loop(0, _D // _NOB, dd_body, 0, unroll=False)

        for q in range(_NOB):
            blk_wait(q)

    return k


def kernel(x, table):
    t0 = table[0]
    r0s = jnp.repeat(t0, 16)
    drs = jnp.repeat(table[1] - t0, 16)
    out3 = _build()(x.T, r0s, drs)
    return jnp.transpose(out3, (2, 0, 1))
